# baseline (speedup 1.0000x reference)
import functools

import jax
import jax.numpy as jnp
from jax import lax
from jax.experimental import pallas as pl
from jax.experimental.pallas import tpu as pltpu
from jax.experimental.pallas import tpu_sc as plsc

NC = 2
NS = 16
L = 16
B = 256
SB = 2
ZR = 200
BN = 2048


def _sc_mesh():
    return plsc.VectorSubcoreMesh(core_axis_name="c", subcore_axis_name="s",
                                  num_cores=NC, num_subcores=NS)


def _tile_batch_range(c, s, nbat):
    w = c * NS + s
    base, rem = nbat // (NC * NS), nbat % (NC * NS)
    b0 = w * base + jnp.minimum(w, rem)
    nb = jnp.where(w < rem, base + 1, base)
    return b0, nb


def _zero_acc(zbuf, acc, s, npad):
    rpt = npad // NS
    for z in range(rpt // ZR):
        pltpu.sync_copy(zbuf, acc.at[pl.ds(s * rpt + z * ZR, ZR)])


def _fill_rows(ref, n, vec):
    def body(i, _):
        ref[i, :] = vec
        return 0
    lax.fori_loop(0, n, body, 0)


def _make_sc_deg(nbat, npad):
    nb = nbat // (NC * NS)
    nsb = nb // SB
    assert nsb % 4 == 0 and nsb >= 8

    @functools.partial(
        pl.kernel,
        out_type=pltpu.HBM((NC, npad, L), jnp.float32),
        mesh=_sc_mesh(),
        compiler_params=pltpu.CompilerParams(use_tc_tiling_on_sc=False),
        scratch_types=[
            pltpu.VMEM((4, SB, B), jnp.int32),
            pltpu.VMEM((B, L), jnp.float32),
            pltpu.VMEM((ZR, L), jnp.float32),
            pltpu.VMEM_SHARED((npad, L), jnp.float32),
            pltpu.SemaphoreType.DMA,
            pltpu.SemaphoreType.DMA,
            pltpu.SemaphoreType.DMA,
        ],
    )
    def deg_kernel(ei, out, didx, ones, zbuf, acc, isem, sem0, sem1):
        c = lax.axis_index("c")
        s = lax.axis_index("s")
        ssem = [sem0, sem1]
        _fill_rows(ones, B, jnp.full((L,), 1.0, jnp.float32))
        _fill_rows(zbuf, ZR, jnp.zeros((L,), jnp.float32))
        _zero_acc(zbuf, acc, s, npad)
        plsc.subcore_barrier()
        b0 = (c * NS + s) * nb
        bmax = b0 + (nsb - 1) * SB

        def idx_copy(q, i):
            base = jnp.minimum(b0 + i * SB, bmax)
            return pltpu.make_async_copy(ei.at[1, pl.ds(base, SB)],
                                         didx.at[q], isem)

        def scatters(p, q):
            return [pltpu.make_async_copy(ones, acc.at[didx.at[q, j]],
                                          ssem[p]) for j in range(SB)]

        def fire_s(p, q):
            for d in scatters(p, q):
                d.start(add=True)

        def wait_s(p, q):
            for d in scatters(p, q):
                d.wait()

        def full_step(i, q4):
            p2 = q4 % 2
            qo = (q4 + 2) % 4
            wait_s(p2, qo)
            idx_copy(qo, i + 2).start()
            idx_copy(q4, i).wait()
            fire_s(p2, q4)

        idx_copy(0, 0).start()
        idx_copy(1, 1).start()
        idx_copy(2, 2).start()
        idx_copy(0, 0).wait()
        fire_s(0, 0)
        idx_copy(3, 3).start()
        idx_copy(1, 1).wait()
        fire_s(1, 1)

        def quad(t, _):
            i = 4 * t
            for q in range(4):
                full_step(i + q - 2, (q + 2) % 4)
            return 0

        lax.fori_loop(1, nsb // 4, quad, 0)
        full_step(nsb - 2, 2)
        full_step(nsb - 1, 3)
        wait_s(0, 2)
        wait_s(1, 3)
        idx_copy(0, nsb).wait()
        idx_copy(1, nsb + 1).wait()
        plsc.subcore_barrier()
        rpt = npad // NS
        pltpu.sync_copy(acc.at[pl.ds(s * rpt, rpt)],
                        out.at[c, pl.ds(s * rpt, rpt)])

    return deg_kernel


def _make_sc_pass(nbat, npad, nchunk):
    assert nchunk in (1, 4)
    kpc = nchunk // NC if nchunk > 1 else 1
    nout = NC if nchunk == 1 else nchunk
    nb = nbat // (NC * NS) if nchunk == 1 else nbat // NS
    nsb = nb // SB
    assert nsb % 4 == 0 and nsb >= 8

    @functools.partial(
        pl.kernel,
        out_type=pltpu.HBM((nout, npad, L), jnp.float32),
        mesh=_sc_mesh(),
        compiler_params=pltpu.CompilerParams(use_tc_tiling_on_sc=False),
        scratch_types=[
            pltpu.VMEM((4, SB, B), jnp.int32),
            pltpu.VMEM((4, SB, B), jnp.int32),
            pltpu.VMEM((2, SB, B, L), jnp.float32),
            pltpu.VMEM((ZR, L), jnp.float32),
            pltpu.VMEM_SHARED((npad, L), jnp.float32),
            pltpu.SemaphoreType.DMA,
            pltpu.SemaphoreType.DMA,
            pltpu.SemaphoreType.DMA,
            pltpu.SemaphoreType.DMA,
            pltpu.SemaphoreType.DMA,
        ],
    )
    def pass_kernel(ei, u, out, sidx, didx, rows, zbuf, acc,
                    isem, gsem0, gsem1, ssem0, ssem1):
        c = lax.axis_index("c")
        s = lax.axis_index("s")
        gsem = [gsem0, gsem1]
        ssem = [ssem0, ssem1]
        _fill_rows(zbuf, ZR, jnp.zeros((L,), jnp.float32))
        rpt = npad // NS

        for kk in range(kpc):
            if nchunk == 1:
                table = u
                out_k = c
                b0 = (c * NS + s) * nb
            else:
                out_k = kpc * c + kk
                table = u.at[out_k]
                b0 = s * nb
            bmax = b0 + (nsb - 1) * SB
            _zero_acc(zbuf, acc, s, npad)
            plsc.subcore_barrier()

            def idx_copies(q, i):
                base = jnp.minimum(b0 + i * SB, bmax)
                return [pltpu.make_async_copy(ei.at[0, pl.ds(base, SB)],
                                              sidx.at[q], isem),
                        pltpu.make_async_copy(ei.at[1, pl.ds(base, SB)],
                                              didx.at[q], isem)]

            def idx_fire(q, i):
                for d in idx_copies(q, i):
                    d.start()

            def idx_wait(q, i):
                for d in idx_copies(q, i):
                    d.wait()

            def gathers(p, q):
                return [pltpu.make_async_copy(table.at[sidx.at[q, j]],
                                              rows.at[p, j], gsem[p])
                        for j in range(SB)]

            def scatters(p, q):
                return [pltpu.make_async_copy(rows.at[p, j],
                                              acc.at[didx.at[q, j]], ssem[p])
                        for j in range(SB)]

            def fire_g(p, q):
                for d in gathers(p, q):
                    d.start()

            def wait_g(p, q):
                for d in gathers(p, q):
                    d.wait()

            def fire_s(p, q):
                for d in scatters(p, q):
                    d.start(add=True)

            def wait_s(p, q):
                for d in scatters(p, q):
                    d.wait()

            def full_step(i, q4):
                p2 = q4 % 2
                qo = (q4 + 2) % 4
                wait_s(p2, qo)
                idx_fire(qo, i + 2)
                idx_wait(q4, i)
                fire_g(p2, q4)
                wait_g(1 - p2, (q4 + 3) % 4)
                fire_s(1 - p2, (q4 + 3) % 4)

            idx_fire(0, 0)
            idx_fire(1, 1)
            idx_fire(2, 2)
            idx_wait(0, 0)
            fire_g(0, 0)
            idx_fire(3, 3)
            idx_wait(1, 1)
            fire_g(1, 1)
            wait_g(0, 0)
            fire_s(0, 0)

            def quad(t, _):
                i = 4 * t
                for q in range(4):
                    full_step(i + q - 2, (q + 2) % 4)
                return 0

            lax.fori_loop(1, nsb // 4, quad, 0)
            full_step(nsb - 2, 2)
            full_step(nsb - 1, 3)
            wait_g(1, 3)
            fire_s(1, 3)
            wait_s(0, 2)
            wait_s(1, 3)
            idx_wait(0, nsb)
            idx_wait(1, nsb + 1)
            plsc.subcore_barrier()
            pltpu.sync_copy(acc.at[pl.ds(s * rpt, rpt)],
                            out.at[out_k, pl.ds(s * rpt, rpt)])
            if kk + 1 < kpc:
                plsc.subcore_barrier()

    return pass_kernel


def _row_specs(npad, shapes):
    specs = []
    for shape in shapes:
        if len(shape) == 3:
            specs.append(pl.BlockSpec((shape[0], BN, shape[2]),
                                      lambda i: (0, i, 0)))
        else:
            specs.append(pl.BlockSpec((BN, shape[1]), lambda i: (i, 0)))
    return specs


def _full_specs(shapes):
    return [pl.BlockSpec(shape, lambda i: tuple(0 for _ in shape))
            for shape in shapes]


def _tc1(npad):
    def body(deg2_ref, x_ref, u1_ref, d16_ref):
        deg = deg2_ref[0, :, 0:1] + deg2_ref[1, :, 0:1] + 1.0
        dinv = lax.rsqrt(deg)
        u1_ref[...] = jnp.concatenate(
            [x_ref[...] * dinv, dinv, jnp.zeros((BN, 7), jnp.float32)], axis=1)
        d16_ref[...] = jnp.broadcast_to(dinv, (BN, L))

    return pl.pallas_call(
        body,
        grid=(npad // BN,),
        in_specs=_row_specs(npad, [(2, npad, L), (npad, 8)]),
        out_specs=_row_specs(npad, [(npad, L), (npad, L)]),
        out_shape=[jax.ShapeDtypeStruct((npad, L), jnp.float32)] * 2,
    )


def _tc2(npad, hid):
    def body(v1_ref, u1_ref, d16_ref, c_ref, w1_ref, b1_ref, u2_ref):
        d = d16_ref[...]
        g = d * (v1_ref[0] + v1_ref[1] + u1_ref[...])
        w1a = w1_ref[0:8, :]
        w1b = w1_ref[8:12, :]
        cw = jnp.dot(c_ref[...], w1b, preferred_element_type=jnp.float32)
        h = jnp.dot(g[:, 0:8], w1a, preferred_element_type=jnp.float32)
        h = jnp.maximum(h + g[:, 8:9] * cw + b1_ref[...], 0.0)
        for kk in range(hid // L):
            u2_ref[kk] = d * h[:, L * kk:L * (kk + 1)]

    return pl.pallas_call(
        body,
        grid=(npad // BN,),
        in_specs=(_row_specs(npad, [(2, npad, L), (npad, L), (npad, L)])
                  + _full_specs([(1, 4), (12, hid), (1, hid)])),
        out_specs=_row_specs(npad, [(hid // L, npad, L)]),
        out_shape=[jax.ShapeDtypeStruct((hid // L, npad, L), jnp.float32)],
    )


def _tc3(npad, hid):
    def body(v2_ref, u2_ref, d16_ref, w2_ref, b2_ref, u3_ref):
        d = d16_ref[...]
        h = jnp.zeros((BN, hid), jnp.float32) + b2_ref[...]
        for kk in range(hid // L):
            gk = d * (v2_ref[kk] + u2_ref[kk])
            h = h + jnp.dot(gk, w2_ref[L * kk:L * (kk + 1), :],
                            preferred_element_type=jnp.float32)
        h = jnp.maximum(h, 0.0)
        for kk in range(hid // L):
            u3_ref[kk] = d * h[:, L * kk:L * (kk + 1)]

    return pl.pallas_call(
        body,
        grid=(npad // BN,),
        in_specs=(_row_specs(npad, [(4, npad, L), (4, npad, L), (npad, L)])
                  + _full_specs([(hid, hid), (1, hid)])),
        out_specs=_row_specs(npad, [(hid // L, npad, L)]),
        out_shape=[jax.ShapeDtypeStruct((hid // L, npad, L), jnp.float32)],
    )


def _tc4(npad, hid, zdim):
    def body(v3_ref, u3_ref, d16_ref, wmu_ref, bmu_ref, wlv_ref, blv_ref,
             mu_ref, lv_ref):
        d = d16_ref[...]
        mu = jnp.zeros((BN, zdim), jnp.float32) + bmu_ref[...]
        lv = jnp.zeros((BN, zdim), jnp.float32) + blv_ref[...]
        for kk in range(hid // L):
            gk = d * (v3_ref[kk] + u3_ref[kk])
            mu = mu + jnp.dot(gk, wmu_ref[L * kk:L * (kk + 1), :],
                              preferred_element_type=jnp.float32)
            lv = lv + jnp.dot(gk, wlv_ref[L * kk:L * (kk + 1), :],
                              preferred_element_type=jnp.float32)
        mu_ref[...] = mu
        lv_ref[...] = lv

    return pl.pallas_call(
        body,
        grid=(npad // BN,),
        in_specs=(_row_specs(npad, [(4, npad, L), (4, npad, L), (npad, L)])
                  + _full_specs([(hid, zdim), (1, zdim),
                                 (hid, zdim), (1, zdim)])),
        out_specs=_row_specs(npad, [(npad, zdim), (npad, zdim)]),
        out_shape=[jax.ShapeDtypeStruct((npad, zdim), jnp.float32)] * 2,
    )


def kernel(x, edge_index, c, W1, b1, W2, b2, Wmu, bmu, Wlv, blv):
    n, in_dim = x.shape
    e = edge_index.shape[1]
    hid = W2.shape[0]
    zdim = Wmu.shape[1]
    assert e % B == 0 and in_dim == 8 and hid == 64
    npad = -(-n // (NS * ZR)) * (NS * ZR)
    assert npad % BN == 0

    ebat = B * SB * NC * NS * 2
    nbat = (-(-e // ebat) * ebat) // B
    epad = nbat * B - e
    ei = jnp.concatenate(
        [edge_index, jnp.full((2, epad), n, jnp.int32)], axis=1
    ).reshape(2, nbat, B)
    xpad = jnp.pad(x, ((0, npad - n), (0, 0)))

    deg2 = _make_sc_deg(nbat, npad)(ei)
    u1, d16 = _tc1(npad)(deg2, xpad)
    v1 = _make_sc_pass(nbat, npad, 1)(ei, u1)
    (u2,) = _tc2(npad, hid)(v1, u1, d16,
                            c.reshape(1, 4), W1, b1.reshape(1, hid))
    sc_pass4 = _make_sc_pass(nbat, npad, 4)
    v2 = sc_pass4(ei, u2)
    (u3,) = _tc3(npad, hid)(v2, u2, d16, W2, b2.reshape(1, hid))
    v3 = sc_pass4(ei, u3)
    mu, lv = _tc4(npad, hid, zdim)(v3, u3, d16,
                                   Wmu, bmu.reshape(1, zdim),
                                   Wlv, blv.reshape(1, zdim))
    return mu[:n], lv[:n]


# speedup vs baseline: 1.1183x; 1.1183x over previous
import functools

import jax
import jax.numpy as jnp
from jax import lax
from jax.experimental import pallas as pl
from jax.experimental.pallas import tpu as pltpu
from jax.experimental.pallas import tpu_sc as plsc

NC = 2
NS = 16
L = 16
B = 256
SB = 1
DR = 4
Q = 8
ZR = 200
BN = 2048


def _sc_mesh():
    return plsc.VectorSubcoreMesh(core_axis_name="c", subcore_axis_name="s",
                                  num_cores=NC, num_subcores=NS)


def _tile_batch_range(c, s, nbat):
    w = c * NS + s
    base, rem = nbat // (NC * NS), nbat % (NC * NS)
    b0 = w * base + jnp.minimum(w, rem)
    nb = jnp.where(w < rem, base + 1, base)
    return b0, nb


def _zero_acc(zbuf, acc, s, npad):
    rpt = npad // NS
    for z in range(rpt // ZR):
        pltpu.sync_copy(zbuf, acc.at[pl.ds(s * rpt + z * ZR, ZR)])


def _fill_rows(ref, n, vec):
    def body(i, _):
        ref[i, :] = vec
        return 0
    lax.fori_loop(0, n, body, 0)


def _make_sc_deg(nbat, npad):
    nb = nbat // (NC * NS)
    nsb = nb // SB
    assert nsb % 4 == 0 and nsb >= 8

    @functools.partial(
        pl.kernel,
        out_type=pltpu.HBM((NC, npad, L), jnp.float32),
        mesh=_sc_mesh(),
        compiler_params=pltpu.CompilerParams(use_tc_tiling_on_sc=False),
        scratch_types=[
            pltpu.VMEM((4, SB, B), jnp.int32),
            pltpu.VMEM((B, L), jnp.float32),
            pltpu.VMEM((ZR, L), jnp.float32),
            pltpu.VMEM_SHARED((npad, L), jnp.float32),
            pltpu.SemaphoreType.DMA,
            pltpu.SemaphoreType.DMA,
            pltpu.SemaphoreType.DMA,
        ],
    )
    def deg_kernel(ei, out, didx, ones, zbuf, acc, isem, sem0, sem1):
        c = lax.axis_index("c")
        s = lax.axis_index("s")
        ssem = [sem0, sem1]
        _fill_rows(ones, B, jnp.full((L,), 1.0, jnp.float32))
        _fill_rows(zbuf, ZR, jnp.zeros((L,), jnp.float32))
        _zero_acc(zbuf, acc, s, npad)
        plsc.subcore_barrier()
        b0 = (c * NS + s) * nb
        bmax = b0 + (nsb - 1) * SB

        def idx_copy(q, i):
            base = jnp.minimum(b0 + i * SB, bmax)
            return pltpu.make_async_copy(ei.at[1, pl.ds(base, SB)],
                                         didx.at[q], isem)

        def scatters(p, q):
            return [pltpu.make_async_copy(ones, acc.at[didx.at[q, j]],
                                          ssem[p]) for j in range(SB)]

        def fire_s(p, q):
            for d in scatters(p, q):
                d.start(add=True)

        def wait_s(p, q):
            for d in scatters(p, q):
                d.wait()

        def full_step(i, q4):
            p2 = q4 % 2
            qo = (q4 + 2) % 4
            wait_s(p2, qo)
            idx_copy(qo, i + 2).start()
            idx_copy(q4, i).wait()
            fire_s(p2, q4)

        idx_copy(0, 0).start()
        idx_copy(1, 1).start()
        idx_copy(2, 2).start()
        idx_copy(0, 0).wait()
        fire_s(0, 0)
        idx_copy(3, 3).start()
        idx_copy(1, 1).wait()
        fire_s(1, 1)

        def quad(t, _):
            i = 4 * t
            for q in range(4):
                full_step(i + q - 2, (q + 2) % 4)
            return 0

        lax.fori_loop(1, nsb // 4, quad, 0)
        full_step(nsb - 2, 2)
        full_step(nsb - 1, 3)
        wait_s(0, 2)
        wait_s(1, 3)
        idx_copy(0, nsb).wait()
        idx_copy(1, nsb + 1).wait()
        plsc.subcore_barrier()
        rpt = npad // NS
        pltpu.sync_copy(acc.at[pl.ds(s * rpt, rpt)],
                        out.at[c, pl.ds(s * rpt, rpt)])

    return deg_kernel


def _make_sc_pass(nbat, npad, nchunk):
    assert nchunk in (1, 4)
    kpc = nchunk // NC if nchunk > 1 else 1
    nout = NC if nchunk == 1 else nchunk
    nb = nbat // (NC * NS) if nchunk == 1 else nbat // NS
    nsb = nb // SB
    assert nsb % Q == 0 and nsb >= 2 * Q

    @functools.partial(
        pl.kernel,
        out_type=pltpu.HBM((nout, npad, L), jnp.float32),
        mesh=_sc_mesh(),
        compiler_params=pltpu.CompilerParams(use_tc_tiling_on_sc=False),
        scratch_types=[
            pltpu.VMEM((Q, SB, B), jnp.int32),
            pltpu.VMEM((Q, SB, B), jnp.int32),
            pltpu.VMEM((DR, SB, B, L), jnp.float32),
            pltpu.VMEM((ZR, L), jnp.float32),
            pltpu.VMEM_SHARED((npad, L), jnp.float32),
            pltpu.SemaphoreType.DMA,
            pltpu.SemaphoreType.DMA,
            pltpu.SemaphoreType.DMA,
            pltpu.SemaphoreType.DMA,
            pltpu.SemaphoreType.DMA,
            pltpu.SemaphoreType.DMA,
            pltpu.SemaphoreType.DMA,
            pltpu.SemaphoreType.DMA,
            pltpu.SemaphoreType.DMA,
        ],
    )
    def pass_kernel(ei, u, out, sidx, didx, rows, zbuf, acc, isem,
                    gsem0, gsem1, gsem2, gsem3, ssem0, ssem1, ssem2, ssem3):
        c = lax.axis_index("c")
        s = lax.axis_index("s")
        gsem = [gsem0, gsem1, gsem2, gsem3]
        ssem = [ssem0, ssem1, ssem2, ssem3]
        _fill_rows(zbuf, ZR, jnp.zeros((L,), jnp.float32))
        rpt = npad // NS

        for kk in range(kpc):
            if nchunk == 1:
                table = u
                out_k = c
                b0 = (c * NS + s) * nb
            else:
                out_k = kpc * c + kk
                table = u.at[out_k]
                b0 = s * nb
            bmax = b0 + (nsb - 1) * SB
            _zero_acc(zbuf, acc, s, npad)
            plsc.subcore_barrier()

            def idx_copies(q, i):
                base = jnp.minimum(b0 + i * SB, bmax)
                return [pltpu.make_async_copy(ei.at[0, pl.ds(base, SB)],
                                              sidx.at[q], isem),
                        pltpu.make_async_copy(ei.at[1, pl.ds(base, SB)],
                                              didx.at[q], isem)]

            def idx_fire(q, i):
                for d in idx_copies(q, i):
                    d.start()

            def idx_wait(q, i):
                for d in idx_copies(q, i):
                    d.wait()

            def gathers(p, q):
                return [pltpu.make_async_copy(table.at[sidx.at[q, j]],
                                              rows.at[p, j], gsem[p])
                        for j in range(SB)]

            def scatters(p, q):
                return [pltpu.make_async_copy(rows.at[p, j],
                                              acc.at[didx.at[q, j]], ssem[p])
                        for j in range(SB)]

            def fire_g(p, q):
                for d in gathers(p, q):
                    d.start()

            def wait_g(p, q):
                for d in gathers(p, q):
                    d.wait()

            def fire_s(p, q):
                for d in scatters(p, q):
                    d.start(add=True)

            def wait_s(p, q):
                for d in scatters(p, q):
                    d.wait()

            def step(i, q8, past_s, past_g):
                pr = q8 % DR
                if past_s:
                    wait_s(pr, q8)
                idx_fire((q8 + 2) % Q, i + 2)
                idx_wait(q8, i)
                fire_g(pr, q8)
                if past_g:
                    qs = (q8 + Q - (DR - 1)) % Q
                    wait_g((pr + 1) % DR, qs)
                    fire_s((pr + 1) % DR, qs)

            idx_fire(0, 0)
            idx_fire(1, 1)
            for i in range(Q):
                step(i, i, i >= DR, i >= DR - 1)

            def octet(t, _):
                i = Q * t
                for q in range(Q):
                    step(i + q, q, True, True)
                return 0

            lax.fori_loop(1, nsb // Q, octet, 0)
            for j in range(DR - 1):
                i = nsb - (DR - 1) + j
                q = i % Q
                wait_g(i % DR, q)
                fire_s(i % DR, q)
            for j in range(DR):
                i = nsb - DR + j
                wait_s(i % DR, i % Q)
            idx_wait(nsb % Q, nsb)
            idx_wait((nsb + 1) % Q, nsb + 1)
            plsc.subcore_barrier()
            pltpu.sync_copy(acc.at[pl.ds(s * rpt, rpt)],
                            out.at[out_k, pl.ds(s * rpt, rpt)])
            if kk + 1 < kpc:
                plsc.subcore_barrier()

    return pass_kernel


def _row_specs(npad, shapes):
    specs = []
    for shape in shapes:
        if len(shape) == 3:
            specs.append(pl.BlockSpec((shape[0], BN, shape[2]),
                                      lambda i: (0, i, 0)))
        else:
            specs.append(pl.BlockSpec((BN, shape[1]), lambda i: (i, 0)))
    return specs


def _full_specs(shapes):
    return [pl.BlockSpec(shape, lambda i: tuple(0 for _ in shape))
            for shape in shapes]


def _tc1(npad):
    def body(deg2_ref, x_ref, u1_ref, d16_ref):
        deg = deg2_ref[0, :, 0:1] + deg2_ref[1, :, 0:1] + 1.0
        dinv = lax.rsqrt(deg)
        u1_ref[...] = jnp.concatenate(
            [x_ref[...] * dinv, dinv, jnp.zeros((BN, 7), jnp.float32)], axis=1)
        d16_ref[...] = jnp.broadcast_to(dinv, (BN, L))

    return pl.pallas_call(
        body,
        grid=(npad // BN,),
        in_specs=_row_specs(npad, [(2, npad, L), (npad, 8)]),
        out_specs=_row_specs(npad, [(npad, L), (npad, L)]),
        out_shape=[jax.ShapeDtypeStruct((npad, L), jnp.float32)] * 2,
    )


def _tc2(npad, hid):
    def body(v1_ref, u1_ref, d16_ref, c_ref, w1_ref, b1_ref, u2_ref):
        d = d16_ref[...]
        g = d * (v1_ref[0] + v1_ref[1] + u1_ref[...])
        w1a = w1_ref[0:8, :]
        w1b = w1_ref[8:12, :]
        cw = jnp.dot(c_ref[...], w1b, preferred_element_type=jnp.float32)
        h = jnp.dot(g[:, 0:8], w1a, preferred_element_type=jnp.float32)
        h = jnp.maximum(h + g[:, 8:9] * cw + b1_ref[...], 0.0)
        for kk in range(hid // L):
            u2_ref[kk] = d * h[:, L * kk:L * (kk + 1)]

    return pl.pallas_call(
        body,
        grid=(npad // BN,),
        in_specs=(_row_specs(npad, [(2, npad, L), (npad, L), (npad, L)])
                  + _full_specs([(1, 4), (12, hid), (1, hid)])),
        out_specs=_row_specs(npad, [(hid // L, npad, L)]),
        out_shape=[jax.ShapeDtypeStruct((hid // L, npad, L), jnp.float32)],
    )


def _tc3(npad, hid):
    def body(v2_ref, u2_ref, d16_ref, w2_ref, b2_ref, u3_ref):
        d = d16_ref[...]
        h = jnp.zeros((BN, hid), jnp.float32) + b2_ref[...]
        for kk in range(hid // L):
            gk = d * (v2_ref[kk] + u2_ref[kk])
            h = h + jnp.dot(gk, w2_ref[L * kk:L * (kk + 1), :],
                            preferred_element_type=jnp.float32)
        h = jnp.maximum(h, 0.0)
        for kk in range(hid // L):
            u3_ref[kk] = d * h[:, L * kk:L * (kk + 1)]

    return pl.pallas_call(
        body,
        grid=(npad // BN,),
        in_specs=(_row_specs(npad, [(4, npad, L), (4, npad, L), (npad, L)])
                  + _full_specs([(hid, hid), (1, hid)])),
        out_specs=_row_specs(npad, [(hid // L, npad, L)]),
        out_shape=[jax.ShapeDtypeStruct((hid // L, npad, L), jnp.float32)],
    )


def _tc4(npad, hid, zdim):
    def body(v3_ref, u3_ref, d16_ref, wmu_ref, bmu_ref, wlv_ref, blv_ref,
             mu_ref, lv_ref):
        d = d16_ref[...]
        mu = jnp.zeros((BN, zdim), jnp.float32) + bmu_ref[...]
        lv = jnp.zeros((BN, zdim), jnp.float32) + blv_ref[...]
        for kk in range(hid // L):
            gk = d * (v3_ref[kk] + u3_ref[kk])
            mu = mu + jnp.dot(gk, wmu_ref[L * kk:L * (kk + 1), :],
                              preferred_element_type=jnp.float32)
            lv = lv + jnp.dot(gk, wlv_ref[L * kk:L * (kk + 1), :],
                              preferred_element_type=jnp.float32)
        mu_ref[...] = mu
        lv_ref[...] = lv

    return pl.pallas_call(
        body,
        grid=(npad // BN,),
        in_specs=(_row_specs(npad, [(4, npad, L), (4, npad, L), (npad, L)])
                  + _full_specs([(hid, zdim), (1, zdim),
                                 (hid, zdim), (1, zdim)])),
        out_specs=_row_specs(npad, [(npad, zdim), (npad, zdim)]),
        out_shape=[jax.ShapeDtypeStruct((npad, zdim), jnp.float32)] * 2,
    )


def kernel(x, edge_index, c, W1, b1, W2, b2, Wmu, bmu, Wlv, blv):
    n, in_dim = x.shape
    e = edge_index.shape[1]
    hid = W2.shape[0]
    zdim = Wmu.shape[1]
    assert e % B == 0 and in_dim == 8 and hid == 64
    npad = -(-n // (NS * ZR)) * (NS * ZR)
    assert npad % BN == 0

    ebat = B * SB * NC * NS * Q
    nbat = (-(-e // ebat) * ebat) // B
    epad = nbat * B - e
    ei = jnp.concatenate(
        [edge_index, jnp.full((2, epad), n, jnp.int32)], axis=1
    ).reshape(2, nbat, B)
    xpad = jnp.pad(x, ((0, npad - n), (0, 0)))

    deg2 = _make_sc_deg(nbat, npad)(ei)
    u1, d16 = _tc1(npad)(deg2, xpad)
    v1 = _make_sc_pass(nbat, npad, 1)(ei, u1)
    (u2,) = _tc2(npad, hid)(v1, u1, d16,
                            c.reshape(1, 4), W1, b1.reshape(1, hid))
    sc_pass4 = _make_sc_pass(nbat, npad, 4)
    v2 = sc_pass4(ei, u2)
    (u3,) = _tc3(npad, hid)(v2, u2, d16, W2, b2.reshape(1, hid))
    v3 = sc_pass4(ei, u3)
    mu, lv = _tc4(npad, hid, zdim)(v3, u3, d16,
                                   Wmu, bmu.reshape(1, zdim),
                                   Wlv, blv.reshape(1, zdim))
    return mu[:n], lv[:n]
